# baseline (speedup 1.0000x reference)
import functools

import jax
import jax.numpy as jnp
from jax import lax
from jax.experimental import pallas as pl
from jax.experimental.pallas import tpu as pltpu
from jax.experimental.pallas import tpu_sc as plsc

_NUM_CORES = 2
_NUM_SUBCORES = 16
_NW = _NUM_CORES * _NUM_SUBCORES
_CHUNK = 16
_NBUF = 3
_LOOK = 2


def kernel(indices, weight):
    b, h = indices.shape
    vocab, d = weight.shape
    n = b * h
    per_w = n // _NW
    nchunk = per_w // _CHUNK
    idx_flat = indices.astype(jnp.int32).T.reshape(n)

    mesh = plsc.VectorSubcoreMesh(core_axis_name="c", subcore_axis_name="s")

    @functools.partial(
        pl.kernel,
        mesh=mesh,
        out_type=jax.ShapeDtypeStruct((n, d), jnp.float32),
        scratch_types=[
            pltpu.VMEM((per_w,), jnp.int32),
            pltpu.VMEM((_NBUF, _CHUNK, d), jnp.float32),
            pltpu.SemaphoreType.DMA,
            pltpu.SemaphoreType.DMA,
            pltpu.SemaphoreType.DMA,
            pltpu.SemaphoreType.DMA,
            pltpu.SemaphoreType.DMA,
            pltpu.SemaphoreType.DMA,
        ],
    )
    def gather_rows(
        idx_hbm, table_hbm, out_hbm, idx_v, rows_v, g0, g1, g2, o0, o1, o2
    ):
        wid = lax.axis_index("s") * _NUM_CORES + lax.axis_index("c")
        base = wid * per_w
        sem_g = (g0, g1, g2)
        sem_o = (o0, o1, o2)
        pltpu.sync_copy(idx_hbm.at[pl.ds(base, per_w)], idx_v)

        def gather_cp(i, buf):
            return pltpu.make_async_copy(
                table_hbm.at[idx_v.at[pl.ds(i * _CHUNK, _CHUNK)]],
                rows_v.at[buf],
                sem_g[buf],
            )

        def out_cp(i, buf):
            return pltpu.make_async_copy(
                rows_v.at[buf],
                out_hbm.at[pl.ds(base + i * _CHUNK, _CHUNK)],
                sem_o[buf],
            )

        for i in range(_LOOK):
            gather_cp(i, i % _NBUF).start()

        def step(i, buf, first, issue_next):
            gather_cp(i, buf).wait()
            out_cp(i, buf).start()
            if issue_next:
                nbuf = (buf + _LOOK) % _NBUF
                if not first:
                    out_cp(i - 1, nbuf).wait()
                gather_cp(i + _LOOK, nbuf).start()

        ngroups = nchunk // _NBUF
        main_chunks = ngroups * _NBUF

        def body(k, carry):
            for c in range(_NBUF):
                step(k * _NBUF + c, c, False, True)
            return carry

        for c in range(_NBUF):
            step(c, c, c == 0, True)
        lax.fori_loop(1, ngroups - 1, body, 0)
        for i in range(main_chunks - _NBUF, nchunk):
            step(i, i % _NBUF, False, i + _LOOK < nchunk)
        for i in range(nchunk - _LOOK - 1, nchunk):
            out_cp(i, i % _NBUF).wait()

    out = gather_rows(idx_flat, weight)
    return out.reshape(h, b, d).transpose(1, 0, 2)


# speedup vs baseline: 1.0025x; 1.0025x over previous
import functools

import jax
import jax.numpy as jnp
from jax import lax
from jax.experimental import pallas as pl
from jax.experimental.pallas import tpu as pltpu
from jax.experimental.pallas import tpu_sc as plsc

_NUM_CORES = 2
_NUM_SUBCORES = 16
_NW = _NUM_CORES * _NUM_SUBCORES
_CHUNK = 24


def kernel(indices, weight):
    b, h = indices.shape
    vocab, d = weight.shape
    n = b * h
    per_w = n // _NW
    nmain = per_w // _CHUNK
    tail = per_w - nmain * _CHUNK
    idx_flat = indices.astype(jnp.int32).T.reshape(n)

    mesh = plsc.VectorSubcoreMesh(core_axis_name="c", subcore_axis_name="s")

    @functools.partial(
        pl.kernel,
        mesh=mesh,
        out_type=jax.ShapeDtypeStruct((n, d), jnp.float32),
        scratch_types=[
            pltpu.VMEM((per_w,), jnp.int32),
            pltpu.VMEM((2, _CHUNK, d), jnp.float32),
            pltpu.SemaphoreType.DMA,
            pltpu.SemaphoreType.DMA,
            pltpu.SemaphoreType.DMA,
            pltpu.SemaphoreType.DMA,
        ],
    )
    def gather_rows(idx_hbm, table_hbm, out_hbm, idx_v, rows_v, g0, g1, o0, o1):
        wid = lax.axis_index("s") * _NUM_CORES + lax.axis_index("c")
        base = wid * per_w
        sem_g = (g0, g1)
        sem_o = (o0, o1)
        pltpu.sync_copy(idx_hbm.at[pl.ds(base, per_w)], idx_v)

        def gather_cp(off, size, buf):
            return pltpu.make_async_copy(
                table_hbm.at[idx_v.at[pl.ds(off, size)]],
                rows_v.at[buf, pl.ds(0, size)],
                sem_g[buf],
            )

        def out_cp(off, size, buf):
            return pltpu.make_async_copy(
                rows_v.at[buf, pl.ds(0, size)],
                out_hbm.at[pl.ds(base + off, size)],
                sem_o[buf],
            )

        gather_cp(0, _CHUNK, 0).start()
        gather_cp(_CHUNK, _CHUNK, 1).start()

        def body(k, carry):
            for c in range(2):
                i = k * 2 + c
                off = i * _CHUNK
                gather_cp(off, _CHUNK, c).wait()
                out_cp(off, _CHUNK, c).start()
                out_cp(off, _CHUNK, c).wait()

                @pl.when(i + 2 < nmain)
                def _():
                    gather_cp(off + 2 * _CHUNK, _CHUNK, c).start()

                if tail and c == nmain % 2:

                    @pl.when(i == nmain - 2)
                    def _():
                        gather_cp(nmain * _CHUNK, tail, c).start()

            return carry

        lax.fori_loop(0, nmain // 2, body, 0)
        if tail:
            buf = nmain % 2
            gather_cp(nmain * _CHUNK, tail, buf).wait()
            out_cp(nmain * _CHUNK, tail, buf).start()
            out_cp(nmain * _CHUNK, tail, buf).wait()

    out = gather_rows(idx_flat, weight)
    return out.reshape(h, b, d).transpose(1, 0, 2)
